# Initial kernel scaffold; baseline (speedup 1.0000x reference)
import jax
import jax.numpy as jnp
from jax.experimental import pallas as pl


def kernel(inputs, shared_weights):
    raise NotImplementedError("write your pallas kernel here")


# speedup vs baseline: 8.4171x; 8.4171x over previous
import jax
import jax.numpy as jnp
from jax.experimental import pallas as pl

HIDDEN = 512
ROWS_PER_BLOCK = 16


def _body(x_ref, w_ref, o_ref):
    x = x_ref[...].astype(jnp.float32)
    w1 = w_ref[1, :] * (HIDDEN ** 0.5)
    o_ref[...] = x[:, :, None] * w1[None, None, :]


def kernel(inputs, shared_weights):
    B, S = inputs.shape
    grid = (B // ROWS_PER_BLOCK,)
    out = pl.pallas_call(
        _body,
        grid=grid,
        in_specs=[
            pl.BlockSpec((ROWS_PER_BLOCK, S), lambda i: (i, 0)),
            pl.BlockSpec((2, HIDDEN), lambda i: (0, 0)),
        ],
        out_specs=pl.BlockSpec((ROWS_PER_BLOCK, S, HIDDEN), lambda i: (i, 0, 0)),
        out_shape=jax.ShapeDtypeStruct((B, S, HIDDEN), jnp.float32),
    )(inputs, shared_weights)
    return out
